# baseline (speedup 1.0000x reference)
import functools

import jax
import jax.numpy as jnp
from jax import lax
from jax.experimental import pallas as pl
from jax.experimental.pallas import tpu as pltpu
from jax.experimental.pallas import tpu_sc as plsc

_B = 1024
_M = 50
_D = 64
_LMEM = 7
_LQ = 19
_HOPS = 2

_NC = 2
_NS = 16
_NW = _NC * _NS

_N1 = _B * _M
_ROWS_W = _N1 // _NW
_CH = 80
_NCH = _ROWS_W // _CH
_TOK = _CH * _LMEM
_SUB = 40
_NSUB = _TOK // _SUB
_QROWS = _B // _NW
_QTOK = _QROWS * _LQ
_QNSUB = (_QTOK + _SUB - 1) // _SUB
_RBUF = _QNSUB * _SUB + 8


def _sc_body(tin, tout, tq, mem_pad, off_ext, q_pad, qoff_ext,
             sum_in, sum_out, sum_q,
             off_v, tok_v, rows_in, rows_out, res_in, res_out, res_q,
             sem_a, sem_b, sem_t):
    wid = lax.axis_index("s") * _NC + lax.axis_index("c")
    row0 = wid * _ROWS_W

    def chunk(ch, carry):
        base = pl.multiple_of(row0 + ch * _CH, 8)
        pltpu.sync_copy(off_ext.at[pl.ds(base, _CH + 8)], off_v.at[pl.ds(0, _CH + 8)])
        t0 = off_v[pl.ds(0, 16)][0]
        base_tok = pl.multiple_of((t0 // 8) * 8, 8)
        pltpu.sync_copy(mem_pad.at[pl.ds(base_tok, _RBUF)], tok_v)
        nrows = off_v[pl.ds(_CH - 8, 16)][8] - base_tok
        cps = [None] * (_NSUB + 1)
        for j in range(_NSUB + 1):
            @pl.when(j * _SUB < nrows)
            def _(j=j):
                cps[j] = (
                    pltpu.async_copy(
                        tin.at[tok_v.at[pl.ds(j * _SUB, _SUB)]],
                        rows_in.at[pl.ds(j * _SUB, _SUB)], sem_a),
                    pltpu.async_copy(
                        tout.at[tok_v.at[pl.ds(j * _SUB, _SUB)]],
                        rows_out.at[pl.ds(j * _SUB, _SUB)], sem_b),
                )
        for j in range(_NSUB + 1):
            @pl.when(j * _SUB < nrows)
            def _(j=j):
                cps[j][0].wait()

        def slot8_in(g, c2):
            va = off_v[pl.ds(g * 8, 16)]
            for i in range(8):
                s = va[i] - base_tok
                ln = va[i + 1] - va[i]
                r = g * 8 + i
                for c in range(_D // 16):
                    acc_i = jnp.zeros((16,), jnp.float32)
                    for jj in range(_LMEM):
                        keep = jj < ln
                        zi = jnp.zeros((16,), jnp.float32)
                        acc_i = acc_i + jnp.where(keep, rows_in[s + jj, pl.ds(c * 16, 16)], zi)
                    res_in[r, pl.ds(c * 16, 16)] = acc_i
            return c2

        lax.fori_loop(0, _CH // 8, slot8_in, 0)
        pltpu.sync_copy(res_in, sum_in.at[pl.ds(base, _CH)])

        for j in range(_NSUB + 1):
            @pl.when(j * _SUB < nrows)
            def _(j=j):
                cps[j][1].wait()

        def slot8_out(g, c2):
            va = off_v[pl.ds(g * 8, 16)]
            for i in range(8):
                s = va[i] - base_tok
                ln = va[i + 1] - va[i]
                r = g * 8 + i
                for c in range(_D // 16):
                    acc_o = jnp.zeros((16,), jnp.float32)
                    for jj in range(_LMEM):
                        keep = jj < ln
                        zi = jnp.zeros((16,), jnp.float32)
                        acc_o = acc_o + jnp.where(keep, rows_out[s + jj, pl.ds(c * 16, 16)], zi)
                    res_out[r, pl.ds(c * 16, 16)] = acc_o
            return c2

        lax.fori_loop(0, _CH // 8, slot8_out, 0)
        pltpu.sync_copy(res_out, sum_out.at[pl.ds(base, _CH)])
        return carry

    lax.fori_loop(0, _NCH, chunk, 0)

    qbase = pl.multiple_of(wid * _QROWS, 8)
    pltpu.sync_copy(qoff_ext.at[pl.ds(qbase, _QROWS + 8)], off_v.at[pl.ds(0, _QROWS + 8)])
    qt0 = off_v[pl.ds(0, 16)][0]
    qbase_tok = pl.multiple_of((qt0 // 8) * 8, 8)
    pltpu.sync_copy(q_pad.at[pl.ds(qbase_tok, _RBUF)], tok_v)
    qnrows = off_v[pl.ds(_QROWS - 8, 16)][8] - qbase_tok
    qcps = [None] * _QNSUB
    for j in range(_QNSUB):
        @pl.when(j * _SUB < qnrows)
        def _(j=j):
            qcps[j] = pltpu.async_copy(
                tq.at[tok_v.at[pl.ds(j * _SUB, _SUB)]],
                rows_in.at[pl.ds(j * _SUB, _SUB)], sem_a)
    for j in range(_QNSUB):
        @pl.when(j * _SUB < qnrows)
        def _(j=j):
            qcps[j].wait()

    def qslot8(g, c2):
        va = off_v[pl.ds(g * 8, 16)]
        for i in range(8):
            s = va[i] - qbase_tok
            ln = va[i + 1] - va[i]
            r = g * 8 + i
            for c in range(_D // 16):
                acc = jnp.zeros((16,), jnp.float32)
                for jj in range(_LQ):
                    acc = acc + jnp.where(jj < ln, rows_in[s + jj, pl.ds(c * 16, 16)],
                                          jnp.zeros((16,), jnp.float32))
                res_q[r, pl.ds(c * 16, 16)] = acc
        return c2

    lax.fori_loop(0, _QROWS // 8, qslot8, 0)
    pltpu.sync_copy(res_q, sum_q.at[pl.ds(qbase, _QROWS)])


def _sc_pool(T_in, T_out, T_query, mem_pad, off_ext, q_pad, qoff_ext):
    f = pl.kernel(
        _sc_body,
        out_type=(
            jax.ShapeDtypeStruct((_N1, _D), jnp.float32),
            jax.ShapeDtypeStruct((_N1, _D), jnp.float32),
            jax.ShapeDtypeStruct((_B, _D), jnp.float32),
        ),
        mesh=plsc.VectorSubcoreMesh(core_axis_name="c", subcore_axis_name="s"),
        scratch_types=[
            pltpu.VMEM((_CH + 24,), jnp.int32),
            pltpu.VMEM((_RBUF,), jnp.int32),
            pltpu.VMEM((_RBUF, _D), jnp.float32),
            pltpu.VMEM(((_NSUB + 1) * _SUB + 8, _D), jnp.float32),
            pltpu.VMEM((_CH, _D), jnp.float32),
            pltpu.VMEM((_CH, _D), jnp.float32),
            pltpu.VMEM((_QROWS, _D), jnp.float32),
            pltpu.SemaphoreType.DMA,
            pltpu.SemaphoreType.DMA,
            pltpu.SemaphoreType.DMA,
        ],
        compiler_params=pltpu.CompilerParams(use_tc_tiling_on_sc=False),
    )
    return f(T_in, T_out, T_query, mem_pad, off_ext, q_pad, qoff_ext)


def _hops_body(ml_ref, ql_ref, w_ref, sin_ref, sout_ref, sq_ref, out_ref):
    f32 = jnp.float32
    ml = ml_ref[...]
    inv_m = 1.0 / jnp.maximum(ml, 1).astype(f32)
    in_mem = sin_ref[...] * inv_m[:, :, None]
    out_mem = sout_ref[...] * inv_m[:, :, None]
    q = sq_ref[...] * (1.0 / jnp.maximum(ql_ref[...], 1).astype(f32))
    w = w_ref[...]
    valid = ml != 0
    neg = jnp.float32(-1e20)
    for _ in range(_HOPS):
        att = jnp.sum(in_mem * q[:, None, :], axis=2)
        att = jnp.where(valid, att, neg)
        att = att - jnp.max(att, axis=1, keepdims=True)
        p = jnp.exp(att)
        p = p / jnp.sum(p, axis=1, keepdims=True)
        mem_out = jnp.sum(p[:, :, None] * out_mem, axis=1)
        q = mem_out + lax.dot_general(q, w, (((1,), (1,)), ((), ())),
                                      preferred_element_type=f32)
    out_ref[...] = q


def _hops(memory_lengths, query_lengths2, W, sum_in, sum_out, sum_q):
    bb = 128
    return pl.pallas_call(
        _hops_body,
        grid=(_B // bb,),
        in_specs=[
            pl.BlockSpec((bb, _M), lambda i: (i, 0)),
            pl.BlockSpec((bb, 1), lambda i: (i, 0)),
            pl.BlockSpec((_D, _D), lambda i: (0, 0)),
            pl.BlockSpec((bb, _M, _D), lambda i: (i, 0, 0)),
            pl.BlockSpec((bb, _M, _D), lambda i: (i, 0, 0)),
            pl.BlockSpec((bb, _D), lambda i: (i, 0)),
        ],
        out_specs=pl.BlockSpec((bb, _D), lambda i: (i, 0)),
        out_shape=jax.ShapeDtypeStruct((_B, _D), jnp.float32),
    )(memory_lengths, query_lengths2, W, sum_in, sum_out, sum_q)


def kernel(memories, queries, memory_lengths, query_lengths, T_query, T_in, T_out, W):
    memories = memories.astype(jnp.int32)
    queries = queries.astype(jnp.int32)

    fl = memory_lengths.reshape(-1).astype(jnp.int32)
    csum = jnp.cumsum(fl)
    off_ext = jnp.concatenate([jnp.zeros((1,), jnp.int32), csum,
                               jnp.full((8,), csum[-1], jnp.int32)])
    mem_pad = jnp.concatenate([memories, jnp.zeros((_RBUF,), jnp.int32)])

    qfl = query_lengths.astype(jnp.int32)
    qcsum = jnp.cumsum(qfl)
    qoff_ext = jnp.concatenate([jnp.zeros((1,), jnp.int32), qcsum,
                                jnp.full((8,), qcsum[-1], jnp.int32)])
    q_pad = jnp.concatenate([queries, jnp.zeros((_RBUF,), jnp.int32)])

    sum_in, sum_out, sum_q = _sc_pool(T_in, T_out, T_query,
                                      mem_pad, off_ext, q_pad, qoff_ext)

    return _hops(
        memory_lengths.astype(jnp.int32),
        qfl[:, None],
        W,
        sum_in.reshape(_B, _M, _D),
        sum_out.reshape(_B, _M, _D),
        sum_q,
    )


# speedup vs baseline: 1.2659x; 1.2659x over previous
import functools

import jax
import jax.numpy as jnp
from jax import lax
from jax.experimental import pallas as pl
from jax.experimental.pallas import tpu as pltpu
from jax.experimental.pallas import tpu_sc as plsc

_B = 1024
_M = 50
_D = 64
_LMEM = 7
_LQ = 19
_HOPS = 2

_NC = 2
_NS = 16
_NW = _NC * _NS

_N1 = _B * _M
_ROWS_W = _N1 // _NW
_CH = 80
_NCH = _ROWS_W // _CH
_TOK = _CH * _LMEM
_SUB = 40
_NSUB = _TOK // _SUB
_QROWS = _B // _NW
_QTOK = _QROWS * _LQ
_QNSUB = (_QTOK + _SUB - 1) // _SUB
_RBUF = _QNSUB * _SUB + 8


def _sc_body(tin, tout, tq, mem_pad, off_ext, q_pad, qoff_ext,
             sum_in, sum_out, sum_q,
             off_v, tok_v, rows_in, rows_out, res_in, res_out, res_q,
             sem_a, sem_b, sem_t):
    wid = lax.axis_index("s") * _NC + lax.axis_index("c")
    row0 = wid * _ROWS_W

    def chunk(ch, carry):
        base = pl.multiple_of(row0 + ch * _CH, 8)
        pltpu.sync_copy(off_ext.at[pl.ds(base, _CH + 8)], off_v.at[pl.ds(0, _CH + 8)])
        t0 = off_v[pl.ds(0, 16)][0]
        base_tok = pl.multiple_of((t0 // 8) * 8, 8)
        pltpu.sync_copy(mem_pad.at[pl.ds(base_tok, _RBUF)], tok_v)
        nrows = off_v[pl.ds(_CH - 8, 16)][8] - base_tok
        cps = [None] * (_NSUB + 1)
        for j in range(_NSUB + 1):
            @pl.when(j * _SUB < nrows)
            def _(j=j):
                cps[j] = (
                    pltpu.async_copy(
                        tin.at[tok_v.at[pl.ds(j * _SUB, _SUB)]],
                        rows_in.at[pl.ds(j * _SUB, _SUB)], sem_a),
                    pltpu.async_copy(
                        tout.at[tok_v.at[pl.ds(j * _SUB, _SUB)]],
                        rows_out.at[pl.ds(j * _SUB, _SUB)], sem_b),
                )
        for j in range(_NSUB + 1):
            @pl.when(j * _SUB < nrows)
            def _(j=j):
                cps[j][0].wait()
                cps[j][1].wait()

        def slot8(g, c2):
            va = off_v[pl.ds(g * 8, 16)]
            for i in range(8):
                s = va[i] - base_tok
                ln = va[i + 1] - va[i]
                r = g * 8 + i
                z = jnp.zeros((16,), jnp.float32)

                def tok(jj, a, s=s):
                    return (
                        tuple(a[c] + rows_in[s + jj, pl.ds(c * 16, 16)]
                              for c in range(_D // 16))
                        + tuple(a[_D // 16 + c] + rows_out[s + jj, pl.ds(c * 16, 16)]
                                for c in range(_D // 16))
                    )

                a = lax.fori_loop(0, ln, tok, (z,) * (2 * (_D // 16)))
                for c in range(_D // 16):
                    res_in[r, pl.ds(c * 16, 16)] = a[c]
                    res_out[r, pl.ds(c * 16, 16)] = a[_D // 16 + c]
            return c2

        lax.fori_loop(0, _CH // 8, slot8, 0)
        pltpu.sync_copy(res_in, sum_in.at[pl.ds(base, _CH)])
        pltpu.sync_copy(res_out, sum_out.at[pl.ds(base, _CH)])
        return carry

    lax.fori_loop(0, _NCH, chunk, 0)

    qbase = pl.multiple_of(wid * _QROWS, 8)
    pltpu.sync_copy(qoff_ext.at[pl.ds(qbase, _QROWS + 8)], off_v.at[pl.ds(0, _QROWS + 8)])
    qt0 = off_v[pl.ds(0, 16)][0]
    qbase_tok = pl.multiple_of((qt0 // 8) * 8, 8)
    pltpu.sync_copy(q_pad.at[pl.ds(qbase_tok, _RBUF)], tok_v)
    qnrows = off_v[pl.ds(_QROWS - 8, 16)][8] - qbase_tok
    qcps = [None] * _QNSUB
    for j in range(_QNSUB):
        @pl.when(j * _SUB < qnrows)
        def _(j=j):
            qcps[j] = pltpu.async_copy(
                tq.at[tok_v.at[pl.ds(j * _SUB, _SUB)]],
                rows_in.at[pl.ds(j * _SUB, _SUB)], sem_a)
    for j in range(_QNSUB):
        @pl.when(j * _SUB < qnrows)
        def _(j=j):
            qcps[j].wait()

    def qslot8(g, c2):
        va = off_v[pl.ds(g * 8, 16)]
        for i in range(8):
            s = va[i] - qbase_tok
            ln = va[i + 1] - va[i]
            r = g * 8 + i
            z = jnp.zeros((16,), jnp.float32)

            def qtok(jj, a, s=s):
                return tuple(a[c] + rows_in[s + jj, pl.ds(c * 16, 16)]
                             for c in range(_D // 16))

            a = lax.fori_loop(0, ln, qtok, (z,) * (_D // 16))
            for c in range(_D // 16):
                res_q[r, pl.ds(c * 16, 16)] = a[c]
        return c2

    lax.fori_loop(0, _QROWS // 8, qslot8, 0)
    pltpu.sync_copy(res_q, sum_q.at[pl.ds(qbase, _QROWS)])


def _sc_pool(T_in, T_out, T_query, mem_pad, off_ext, q_pad, qoff_ext):
    f = pl.kernel(
        _sc_body,
        out_type=(
            jax.ShapeDtypeStruct((_N1, _D), jnp.float32),
            jax.ShapeDtypeStruct((_N1, _D), jnp.float32),
            jax.ShapeDtypeStruct((_B, _D), jnp.float32),
        ),
        mesh=plsc.VectorSubcoreMesh(core_axis_name="c", subcore_axis_name="s"),
        scratch_types=[
            pltpu.VMEM((_CH + 24,), jnp.int32),
            pltpu.VMEM((_RBUF,), jnp.int32),
            pltpu.VMEM((_RBUF, _D), jnp.float32),
            pltpu.VMEM(((_NSUB + 1) * _SUB + 8, _D), jnp.float32),
            pltpu.VMEM((_CH, _D), jnp.float32),
            pltpu.VMEM((_CH, _D), jnp.float32),
            pltpu.VMEM((_QROWS, _D), jnp.float32),
            pltpu.SemaphoreType.DMA,
            pltpu.SemaphoreType.DMA,
            pltpu.SemaphoreType.DMA,
        ],
        compiler_params=pltpu.CompilerParams(use_tc_tiling_on_sc=False),
    )
    return f(T_in, T_out, T_query, mem_pad, off_ext, q_pad, qoff_ext)


def _hops_body(ml_ref, ql_ref, w_ref, sin_ref, sout_ref, sq_ref, out_ref):
    f32 = jnp.float32
    ml = ml_ref[...]
    inv_m = 1.0 / jnp.maximum(ml, 1).astype(f32)
    in_mem = sin_ref[...] * inv_m[:, :, None]
    out_mem = sout_ref[...] * inv_m[:, :, None]
    q = sq_ref[...] * (1.0 / jnp.maximum(ql_ref[...], 1).astype(f32))
    w = w_ref[...]
    valid = ml != 0
    neg = jnp.float32(-1e20)
    for _ in range(_HOPS):
        att = jnp.sum(in_mem * q[:, None, :], axis=2)
        att = jnp.where(valid, att, neg)
        att = att - jnp.max(att, axis=1, keepdims=True)
        p = jnp.exp(att)
        p = p / jnp.sum(p, axis=1, keepdims=True)
        mem_out = jnp.sum(p[:, :, None] * out_mem, axis=1)
        q = mem_out + lax.dot_general(q, w, (((1,), (1,)), ((), ())),
                                      preferred_element_type=f32)
    out_ref[...] = q


def _hops(memory_lengths, query_lengths2, W, sum_in, sum_out, sum_q):
    bb = 128
    return pl.pallas_call(
        _hops_body,
        grid=(_B // bb,),
        in_specs=[
            pl.BlockSpec((bb, _M), lambda i: (i, 0)),
            pl.BlockSpec((bb, 1), lambda i: (i, 0)),
            pl.BlockSpec((_D, _D), lambda i: (0, 0)),
            pl.BlockSpec((bb, _M, _D), lambda i: (i, 0, 0)),
            pl.BlockSpec((bb, _M, _D), lambda i: (i, 0, 0)),
            pl.BlockSpec((bb, _D), lambda i: (i, 0)),
        ],
        out_specs=pl.BlockSpec((bb, _D), lambda i: (i, 0)),
        out_shape=jax.ShapeDtypeStruct((_B, _D), jnp.float32),
    )(memory_lengths, query_lengths2, W, sum_in, sum_out, sum_q)


def kernel(memories, queries, memory_lengths, query_lengths, T_query, T_in, T_out, W):
    memories = memories.astype(jnp.int32)
    queries = queries.astype(jnp.int32)

    fl = memory_lengths.reshape(-1).astype(jnp.int32)
    csum = jnp.cumsum(fl)
    off_ext = jnp.concatenate([jnp.zeros((1,), jnp.int32), csum,
                               jnp.full((8,), csum[-1], jnp.int32)])
    mem_pad = jnp.concatenate([memories, jnp.zeros((_RBUF,), jnp.int32)])

    qfl = query_lengths.astype(jnp.int32)
    qcsum = jnp.cumsum(qfl)
    qoff_ext = jnp.concatenate([jnp.zeros((1,), jnp.int32), qcsum,
                                jnp.full((8,), qcsum[-1], jnp.int32)])
    q_pad = jnp.concatenate([queries, jnp.zeros((_RBUF,), jnp.int32)])

    sum_in, sum_out, sum_q = _sc_pool(T_in, T_out, T_query,
                                      mem_pad, off_ext, q_pad, qoff_ext)

    return _hops(
        memory_lengths.astype(jnp.int32),
        qfl[:, None],
        W,
        sum_in.reshape(_B, _M, _D),
        sum_out.reshape(_B, _M, _D),
        sum_q,
    )
